# Initial kernel scaffold; baseline (speedup 1.0000x reference)
import functools

import jax
import jax.numpy as jnp
from jax import lax
from jax.experimental import pallas as pl
from jax.experimental.pallas import tpu as pltpu
from jax.experimental.pallas import tpu_sc as plsc

_N = 10000
_D = 128
_NC = 2
_NS = 16
_NW = _NC * _NS
_CHUNK = 128
_N_PAD = 10240


def _sc_aggregate(X, src_r, dst_r, zeros):
    K = src_r.shape[2]
    mesh = plsc.VectorSubcoreMesh(core_axis_name="c", subcore_axis_name="s")

    @functools.partial(
        pl.kernel,
        mesh=mesh,
        out_type=jax.ShapeDtypeStruct((_NC, _N, _D), jnp.float32),
        scratch_types=[
            pltpu.VMEM((K, _CHUNK), jnp.int32),
            pltpu.VMEM((K, _CHUNK), jnp.int32),
            pltpu.VMEM((_CHUNK, _D), jnp.float32),
            pltpu.VMEM_SHARED((_N_PAD, _D), jnp.float32),
        ],
    )
    def agg_kernel(x_hbm, src_hbm, dst_hbm, zeros_hbm, out_hbm,
                   src_v, dst_v, rows_v, acc_sh):
        cid = lax.axis_index("c")
        sid = lax.axis_index("s")

        zrows = _N_PAD // _NS
        pltpu.sync_copy(zeros_hbm.at[pl.ds(sid * zrows, zrows)],
                        acc_sh.at[pl.ds(sid * zrows, zrows)])
        pltpu.sync_copy(src_hbm.at[cid, sid], src_v)
        pltpu.sync_copy(dst_hbm.at[cid, sid], dst_v)
        plsc.subcore_barrier()

        @pl.loop(0, K)
        def _(j):
            pltpu.sync_copy(x_hbm.at[src_v.at[j]], rows_v)
            pltpu.sync_copy(rows_v, acc_sh.at[dst_v.at[j]], add=True)

        plsc.subcore_barrier()
        orows = _N // _NS
        pltpu.sync_copy(acc_sh.at[pl.ds(sid * orows, orows)],
                        out_hbm.at[cid, pl.ds(sid * orows, orows)])

    return agg_kernel(X, src_r, dst_r, zeros)


def _tc_finish(p0, p1, X, W):
    blk = 1000

    def mm_kernel(p0_ref, p1_ref, x_ref, w_ref, o_ref):
        agg = p0_ref[...] + p1_ref[...] + x_ref[...]
        o_ref[...] = jnp.dot(agg, w_ref[...],
                             preferred_element_type=jnp.float32)

    row_spec = pl.BlockSpec((blk, _D), lambda i: (i, 0))
    return pl.pallas_call(
        mm_kernel,
        grid=(_N // blk,),
        in_specs=[row_spec, row_spec, row_spec,
                  pl.BlockSpec((_D, _D), lambda i: (0, 0))],
        out_specs=row_spec,
        out_shape=jax.ShapeDtypeStruct((_N, _D), jnp.float32),
    )(p0, p1, X, W)


def kernel(X, edge_index, W):
    src = edge_index[0]
    dst = edge_index[1]
    E = src.shape[0]

    K = -(-E // (_NW * _CHUNK))
    e_pad = _NW * K * _CHUNK
    pad = e_pad - E
    src_p = jnp.concatenate([src, jnp.zeros((pad,), jnp.int32)])
    dst_p = jnp.concatenate(
        [dst, _N + (jnp.arange(pad, dtype=jnp.int32) % (_N_PAD - _N))])
    src_r = src_p.reshape(_NC, _NS, K, _CHUNK)
    dst_r = dst_p.reshape(_NC, _NS, K, _CHUNK)
    zeros = jnp.zeros((_N_PAD, _D), jnp.float32)

    partials = _sc_aggregate(X, src_r, dst_r, zeros)
    return _tc_finish(partials[0], partials[1], X, W)


# speedup vs baseline: 4.7722x; 4.7722x over previous
import functools

import jax
import jax.numpy as jnp
from jax import lax
from jax.experimental import pallas as pl
from jax.experimental.pallas import tpu as pltpu
from jax.experimental.pallas import tpu_sc as plsc

_N = 10000
_D = 128
_NC = 2
_NS = 16
_NW = _NC * _NS
_CHUNK = 128
_N_PAD = 10240


def _sc_aggregate(X, src_r, dst_r, zeros):
    K = src_r.shape[2]
    mesh = plsc.VectorSubcoreMesh(core_axis_name="c", subcore_axis_name="s")

    @functools.partial(
        pl.kernel,
        mesh=mesh,
        out_type=jax.ShapeDtypeStruct((_NC, _N_PAD, _D), jnp.float32),
        scratch_types=[
            pltpu.VMEM((K, _CHUNK), jnp.int32),
            pltpu.VMEM((K, _CHUNK), jnp.int32),
            pltpu.VMEM((_CHUNK, _D), jnp.float32),
            pltpu.VMEM_SHARED((_N_PAD, _D), jnp.float32),
        ],
    )
    def agg_kernel(x_hbm, src_hbm, dst_hbm, zeros_hbm, out_hbm,
                   src_v, dst_v, rows_v, acc_sh):
        cid = lax.axis_index("c")
        sid = lax.axis_index("s")

        zrows = _N_PAD // _NS
        pltpu.sync_copy(zeros_hbm.at[pl.ds(sid * zrows, zrows)],
                        acc_sh.at[pl.ds(sid * zrows, zrows)])
        pltpu.sync_copy(src_hbm.at[cid, sid], src_v)
        pltpu.sync_copy(dst_hbm.at[cid, sid], dst_v)
        plsc.subcore_barrier()

        @pl.loop(0, K)
        def _(j):
            pltpu.sync_copy(x_hbm.at[src_v.at[j]], rows_v)
            pltpu.sync_copy(rows_v, acc_sh.at[dst_v.at[j]], add=True)

        plsc.subcore_barrier()
        orows = _N_PAD // _NS
        pltpu.sync_copy(acc_sh.at[pl.ds(sid * orows, orows)],
                        out_hbm.at[cid, pl.ds(sid * orows, orows)])

    return agg_kernel(X, src_r, dst_r, zeros)


def _tc_finish(p0, p1, X, W):
    blk = 1000

    def mm_kernel(p0_ref, p1_ref, x_ref, w_ref, o_ref):
        agg = p0_ref[...] + p1_ref[...] + x_ref[...]
        o_ref[...] = jnp.dot(agg, w_ref[...],
                             preferred_element_type=jnp.float32)

    row_spec = pl.BlockSpec((blk, _D), lambda i: (i, 0))
    return pl.pallas_call(
        mm_kernel,
        grid=(_N // blk,),
        in_specs=[row_spec, row_spec, row_spec,
                  pl.BlockSpec((_D, _D), lambda i: (0, 0))],
        out_specs=row_spec,
        out_shape=jax.ShapeDtypeStruct((_N, _D), jnp.float32),
    )(p0, p1, X, W)


def kernel(X, edge_index, W):
    src = edge_index[0]
    dst = edge_index[1]
    E = src.shape[0]

    K = -(-E // (_NW * _CHUNK))
    e_pad = _NW * K * _CHUNK
    pad = e_pad - E
    src_p = jnp.concatenate([src, jnp.zeros((pad,), jnp.int32)])
    dst_p = jnp.concatenate(
        [dst, _N + (jnp.arange(pad, dtype=jnp.int32) % (_N_PAD - _N))])
    src_r = src_p.reshape(_NC, _NS, K, _CHUNK)
    dst_r = dst_p.reshape(_NC, _NS, K, _CHUNK)
    zeros = jnp.zeros((_N_PAD, _D), jnp.float32)

    partials = _sc_aggregate(X, src_r, dst_r, zeros)
    return _tc_finish(partials[0], partials[1], X, W)
